# baseline (speedup 1.0000x reference)
import jax
import jax.numpy as jnp
from jax import lax
from jax.experimental import pallas as pl
from jax.experimental.pallas import tpu as pltpu
from jax.experimental.pallas import tpu_sc as plsc

L = 16
NC = 2
NS = 16
DMAX = 8


def _splat(ref1d, pos):
    return plsc.load_gather(ref1d, [jnp.full((L,), pos, jnp.int32)])


def _cvr_body(sizes, n_num, cpad, poffs, bpw,
              xnT_hbm, w_hbm, b_hbm, tT_hbm, widx_hbm, out_hbm,
              x_v, tt_v, w_v, b_v, widx_v, cseg_v, c_v, acc_v,
              shc, sem_fold, sem_in):
    cid = lax.axis_index("c")
    sid = lax.axis_index("s")
    wid = sid * NC + cid
    base = wid * bpw
    cseg = cpad // NS
    nchk = cseg // L

    h_t = pltpu.async_copy(tT_hbm.at[:, pl.ds(sid * cseg, cseg)], tt_v,
                           sem_fold)
    h_w = pltpu.async_copy(w_hbm, w_v, sem_fold)
    h_wi = pltpu.async_copy(widx_hbm.at[pl.ds(sid * cseg, cseg)], widx_v,
                            sem_fold)
    h_x = pltpu.async_copy(xnT_hbm.at[:, pl.ds(base, bpw)], x_v, sem_in)
    h_b = pltpu.async_copy(b_hbm, b_v, sem_in)

    h_t.wait()
    h_w.wait()
    h_wi.wait()
    for cl in range(nchk):
        sl = pl.ds(cl * L, L)
        wbase = widx_v[sl]
        acc = tt_v[0, sl] * plsc.load_gather(w_v, [wbase])
        for dd in range(1, DMAX):
            acc = acc + tt_v[dd, sl] * plsc.load_gather(w_v, [wbase + dd])
        cseg_v[sl] = acc

    pltpu.sync_copy(cseg_v, shc.at[pl.ds(sid * cseg, cseg)])
    plsc.subcore_barrier()
    pltpu.sync_copy(shc, c_v)

    h_x.wait()
    h_b.wait()
    wnum = [_splat(w_v, 100 + k) for k in range(n_num)]
    bias = _splat(b_v, 0)
    nf = len(sizes)

    def chunk(ch, carry):
        sl = pl.ds(ch * L, L)
        acc = bias
        for k in range(n_num):
            num = plsc.bitcast(x_v[nf + k, sl], jnp.float32)
            acc = acc + num * wnum[k]
        for i in range(len(sizes)):
            idx = x_v[i, sl] + poffs[i]
            acc = acc + plsc.load_gather(c_v, [idx])
        acc_v[sl] = acc
        return carry

    lax.fori_loop(0, bpw // L, chunk, 0)
    pltpu.sync_copy(acc_v, out_hbm.at[pl.ds(base, bpw)])


def kernel(x, numerical_feature, W, b,
           t0, t1, t2, t3, t4, t5, t6, t7, t8, t9, t10, t11, t12, t13, t14):
    tables = [t0, t1, t2, t3, t4, t5, t6, t7, t8, t9, t10, t11, t12, t13, t14]
    sizes = tuple((t.shape[0], t.shape[1]) for t in tables)
    B = x.shape[0]
    n_num = numerical_feature.shape[1]
    bpw = B // (NC * NS)
    assert B % (NC * NS * L) == 0

    poffs, off = [], 0
    for n, _ in sizes:
        poffs.append(off)
        off += -(-n // L) * L
    cpad = -(-off // (NS * 128)) * (NS * 128)

    pieces = []
    for t, (n, d) in zip(tables, sizes):
        if d < DMAX:
            t = jnp.concatenate([t, jnp.zeros((n, DMAX - d), jnp.float32)], 1)
        pieces.append(t)
        rowpad = (-(-n // L) * L) - n
        if rowpad:
            pieces.append(jnp.zeros((rowpad, DMAX), jnp.float32))
    if cpad > off:
        pieces.append(jnp.zeros((cpad - off, DMAX), jnp.float32))
    tT = jnp.concatenate(pieces, 0).T
    xnT = jnp.concatenate(
        [x.astype(jnp.int32),
         lax.bitcast_convert_type(numerical_feature, jnp.int32)], 1).T
    Wf = jnp.pad(W[:, 0], (0, -W.shape[0] % 8))

    import numpy as np
    woffs, woff = [], 0
    for _, d in sizes:
        woffs.append(woff)
        woff += d
    widxmap = np.zeros((cpad,), np.int32)
    for i, (n, _) in enumerate(sizes):
        slot = -(-n // L) * L
        widxmap[poffs[i]:poffs[i] + slot] = woffs[i]
    widxmap = jnp.asarray(widxmap)

    def body(*refs):
        _cvr_body(sizes, n_num, cpad, poffs, bpw, *refs)

    run = pl.kernel(
        body,
        out_type=jax.ShapeDtypeStruct((B,), jnp.float32),
        mesh=plsc.VectorSubcoreMesh(core_axis_name="c", subcore_axis_name="s",
                                    num_cores=NC, num_subcores=NS),
        compiler_params=pltpu.CompilerParams(needs_layout_passes=False),
        scratch_types=[
            pltpu.VMEM((len(sizes) + n_num, bpw), jnp.int32),
            pltpu.VMEM((DMAX, cpad // NS), jnp.float32),
            pltpu.VMEM((W.shape[0] + (-W.shape[0] % 8),), jnp.float32),
            pltpu.VMEM(b.shape, jnp.float32),
            pltpu.VMEM((cpad // NS,), jnp.int32),
            pltpu.VMEM((cpad // NS,), jnp.float32),
            pltpu.VMEM((cpad,), jnp.float32),
            pltpu.VMEM((bpw,), jnp.float32),
            pltpu.VMEM_SHARED((cpad,), jnp.float32),
            pltpu.SemaphoreType.DMA,
            pltpu.SemaphoreType.DMA,
        ],
    )
    out = run(xnT, Wf, b, tT, widxmap)
    return out.reshape(B, 1)


# speedup vs baseline: 1.1441x; 1.1441x over previous
import jax
import jax.numpy as jnp
from jax import lax
from jax.experimental import pallas as pl
from jax.experimental.pallas import tpu as pltpu
from jax.experimental.pallas import tpu_sc as plsc

L = 16
NC = 2
NS = 16
DMAX = 8


def _splat(ref1d, pos):
    return plsc.load_gather(ref1d, [jnp.full((L,), pos, jnp.int32)])


def _cvr_body(sizes, n_num, cpad, poffs, bpw,
              xT_hbm, numT_hbm, w_hbm, b_hbm, tT_hbm, widx_hbm, out_hbm,
              x_v, num_v, tt_v, w_v, b_v, widx_v, cseg_v, c_v, acc_v,
              shc, sem_fold, sem_in):
    cid = lax.axis_index("c")
    sid = lax.axis_index("s")
    wid = sid * NC + cid
    base = wid * bpw
    cseg = cpad // NS
    nchk = cseg // L

    h_t = pltpu.async_copy(tT_hbm.at[:, pl.ds(sid * cseg, cseg)], tt_v,
                           sem_fold)
    h_w = pltpu.async_copy(w_hbm, w_v, sem_fold)
    h_wi = pltpu.async_copy(widx_hbm.at[pl.ds(sid * cseg, cseg)], widx_v,
                            sem_fold)
    h_x = pltpu.async_copy(xT_hbm.at[:, pl.ds(base, bpw)], x_v, sem_in)
    h_n = pltpu.async_copy(numT_hbm.at[:, pl.ds(base, bpw)], num_v, sem_in)
    h_b = pltpu.async_copy(b_hbm, b_v, sem_in)

    h_t.wait()
    h_w.wait()
    h_wi.wait()
    for cl in range(nchk):
        sl = pl.ds(cl * L, L)
        wbase = widx_v[sl]
        acc = tt_v[0, sl] * plsc.load_gather(w_v, [wbase])
        for dd in range(1, DMAX):
            acc = acc + tt_v[dd, sl] * plsc.load_gather(w_v, [wbase + dd])
        cseg_v[sl] = acc

    pltpu.sync_copy(cseg_v, shc.at[pl.ds(sid * cseg, cseg)])
    plsc.subcore_barrier()
    pltpu.sync_copy(shc, c_v)

    h_x.wait()
    h_n.wait()
    h_b.wait()
    wnum = [_splat(w_v, 100 + k) for k in range(n_num)]
    bias = _splat(b_v, 0)

    def chunk(ch, carry):
        sl = pl.ds(ch * L, L)
        acc = bias
        for k in range(n_num):
            acc = acc + num_v[k, sl] * wnum[k]
        for i in range(len(sizes)):
            idx = x_v[i, sl] + poffs[i]
            acc = acc + plsc.load_gather(c_v, [idx])
        acc_v[sl] = acc
        return carry

    lax.fori_loop(0, bpw // L, chunk, 0)
    pltpu.sync_copy(acc_v, out_hbm.at[pl.ds(base, bpw)])


def kernel(x, numerical_feature, W, b,
           t0, t1, t2, t3, t4, t5, t6, t7, t8, t9, t10, t11, t12, t13, t14):
    tables = [t0, t1, t2, t3, t4, t5, t6, t7, t8, t9, t10, t11, t12, t13, t14]
    sizes = tuple((t.shape[0], t.shape[1]) for t in tables)
    B = x.shape[0]
    n_num = numerical_feature.shape[1]
    bpw = B // (NC * NS)
    assert B % (NC * NS * L) == 0

    poffs, off = [], 0
    for n, _ in sizes:
        poffs.append(off)
        off += -(-n // L) * L
    cpad = -(-off // (NS * 128)) * (NS * 128)

    pieces = []
    for t, (n, d) in zip(tables, sizes):
        if d < DMAX:
            t = jnp.concatenate([t, jnp.zeros((n, DMAX - d), jnp.float32)], 1)
        pieces.append(t)
        rowpad = (-(-n // L) * L) - n
        if rowpad:
            pieces.append(jnp.zeros((rowpad, DMAX), jnp.float32))
    if cpad > off:
        pieces.append(jnp.zeros((cpad - off, DMAX), jnp.float32))
    tT = jnp.concatenate(pieces, 0).T
    xT = x.astype(jnp.int32).T
    numT = numerical_feature.T
    Wf = jnp.pad(W[:, 0], (0, -W.shape[0] % 8))

    import numpy as np
    woffs, woff = [], 0
    for _, d in sizes:
        woffs.append(woff)
        woff += d
    widxmap = np.zeros((cpad,), np.int32)
    for i, (n, _) in enumerate(sizes):
        slot = -(-n // L) * L
        widxmap[poffs[i]:poffs[i] + slot] = woffs[i]
    widxmap = jnp.asarray(widxmap)

    def body(*refs):
        _cvr_body(sizes, n_num, cpad, poffs, bpw, *refs)

    run = pl.kernel(
        body,
        out_type=jax.ShapeDtypeStruct((B,), jnp.float32),
        mesh=plsc.VectorSubcoreMesh(core_axis_name="c", subcore_axis_name="s",
                                    num_cores=NC, num_subcores=NS),
        compiler_params=pltpu.CompilerParams(needs_layout_passes=False),
        scratch_types=[
            pltpu.VMEM((len(sizes), bpw), jnp.int32),
            pltpu.VMEM((n_num, bpw), jnp.float32),
            pltpu.VMEM((DMAX, cpad // NS), jnp.float32),
            pltpu.VMEM((W.shape[0] + (-W.shape[0] % 8),), jnp.float32),
            pltpu.VMEM(b.shape, jnp.float32),
            pltpu.VMEM((cpad // NS,), jnp.int32),
            pltpu.VMEM((cpad // NS,), jnp.float32),
            pltpu.VMEM((cpad,), jnp.float32),
            pltpu.VMEM((bpw,), jnp.float32),
            pltpu.VMEM_SHARED((cpad,), jnp.float32),
            pltpu.SemaphoreType.DMA,
            pltpu.SemaphoreType.DMA,
        ],
    )
    out = run(xT, numT, Wf, b, tT, widxmap)
    return out.reshape(B, 1)
